# Initial kernel scaffold; baseline (speedup 1.0000x reference)
import jax
import jax.numpy as jnp
from jax.experimental import pallas as pl


def kernel(x, edge_index, W_l1, W_r1, b1, g1, be1, Wg, att_s, att_d, bg, g2, be2, W_l3, W_r3, b3, g3, be3, Ws, bs, Wc1, bc1, Wc2, bc2):
    raise NotImplementedError("write your pallas kernel here")


# speedup vs baseline: 2.5348x; 2.5348x over previous
import jax
import jax.numpy as jnp
from jax.experimental import pallas as pl
from jax.experimental.pallas import tpu as pltpu

_N = 50000
_E = 800000
_IN = 20
_H = 64
_HEADS = 4

_ARB = pltpu.CompilerParams(dimension_semantics=("arbitrary",))


def _scatter(dst, m, n_rows, op):
    e, d = m.shape
    chunk = 256
    init = 0.0 if op == "add" else -jnp.inf

    def kern(dst_ref, m_ref, out_ref):
        @pl.when(pl.program_id(0) == 0)
        def _():
            out_ref[...] = jnp.full_like(out_ref, init)

        def body(j, carry):
            r = dst_ref[j]
            row = m_ref[pl.ds(j, 1), :]
            cur = out_ref[pl.ds(r, 1), :]
            new = cur + row if op == "add" else jnp.maximum(cur, row)
            out_ref[pl.ds(r, 1), :] = new
            return carry

        jax.lax.fori_loop(0, chunk, body, 0)

    return pl.pallas_call(
        kern,
        grid=(e // chunk,),
        in_specs=[
            pl.BlockSpec((chunk,), lambda i: (i,), memory_space=pltpu.SMEM),
            pl.BlockSpec((chunk, d), lambda i: (i, 0)),
        ],
        out_specs=pl.BlockSpec((n_rows, d), lambda i: (0, 0)),
        out_shape=jax.ShapeDtypeStruct((n_rows, d), jnp.float32),
        compiler_params=_ARB,
    )(dst, m)


def _dense(a, w, b, act=None):
    n, k = a.shape
    h = w.shape[1]
    blk = 2000

    def kern(a_ref, w_ref, b_ref, o_ref):
        o = jnp.dot(a_ref[...], w_ref[...], preferred_element_type=jnp.float32)
        o = o + b_ref[...]
        if act == "relu":
            o = jnp.maximum(o, 0.0)
        o_ref[...] = o

    return pl.pallas_call(
        kern,
        grid=(n // blk,),
        in_specs=[
            pl.BlockSpec((blk, k), lambda i: (i, 0)),
            pl.BlockSpec((k, h), lambda i: (0, 0)),
            pl.BlockSpec((1, h), lambda i: (0, 0)),
        ],
        out_specs=pl.BlockSpec((blk, h), lambda i: (i, 0)),
        out_shape=jax.ShapeDtypeStruct((n, h), jnp.float32),
    )(a, w, b.reshape(1, -1))


def _dense2(a, wa, c, wc, b):
    n, ka = a.shape
    kc = c.shape[1]
    h = wa.shape[1]
    blk = 2000

    def kern(a_ref, wa_ref, c_ref, wc_ref, b_ref, o_ref):
        o = jnp.dot(a_ref[...], wa_ref[...], preferred_element_type=jnp.float32)
        o = o + jnp.dot(c_ref[...], wc_ref[...], preferred_element_type=jnp.float32)
        o_ref[...] = o + b_ref[...]

    return pl.pallas_call(
        kern,
        grid=(n // blk,),
        in_specs=[
            pl.BlockSpec((blk, ka), lambda i: (i, 0)),
            pl.BlockSpec((ka, h), lambda i: (0, 0)),
            pl.BlockSpec((blk, kc), lambda i: (i, 0)),
            pl.BlockSpec((kc, h), lambda i: (0, 0)),
            pl.BlockSpec((1, h), lambda i: (0, 0)),
        ],
        out_specs=pl.BlockSpec((blk, h), lambda i: (i, 0)),
        out_shape=jax.ShapeDtypeStruct((n, h), jnp.float32),
    )(a, wa, c, wc, b.reshape(1, -1))


def _bn_stats(h):
    n, d = h.shape
    blk = 2000

    def kern(h_ref, s_ref, q_ref):
        @pl.when(pl.program_id(0) == 0)
        def _():
            s_ref[...] = jnp.zeros_like(s_ref)
            q_ref[...] = jnp.zeros_like(q_ref)

        x = h_ref[...]
        s_ref[...] += jnp.sum(x, axis=0, keepdims=True)
        q_ref[...] += jnp.sum(x * x, axis=0, keepdims=True)

    s, q = pl.pallas_call(
        kern,
        grid=(n // blk,),
        in_specs=[pl.BlockSpec((blk, d), lambda i: (i, 0))],
        out_specs=[pl.BlockSpec((1, d), lambda i: (0, 0))] * 2,
        out_shape=[jax.ShapeDtypeStruct((1, d), jnp.float32)] * 2,
        compiler_params=_ARB,
    )(h)
    m = s / n
    v = q / n - m * m
    return m, v


def _affine_relu(h, scale, shift):
    n, d = h.shape
    blk = 2000

    def kern(h_ref, sc_ref, sh_ref, o_ref):
        o_ref[...] = jnp.maximum(h_ref[...] * sc_ref[...] + sh_ref[...], 0.0)

    return pl.pallas_call(
        kern,
        grid=(n // blk,),
        in_specs=[
            pl.BlockSpec((blk, d), lambda i: (i, 0)),
            pl.BlockSpec((1, d), lambda i: (0, 0)),
            pl.BlockSpec((1, d), lambda i: (0, 0)),
        ],
        out_specs=pl.BlockSpec((blk, d), lambda i: (i, 0)),
        out_shape=jax.ShapeDtypeStruct((n, d), jnp.float32),
    )(h, scale, shift)


def _bn_relu(h, g, be):
    m, v = _bn_stats(h)
    scale = g.reshape(1, -1) / jnp.sqrt(v + 1e-5)
    shift = be.reshape(1, -1) - m * scale
    return _affine_relu(h, scale, shift)


def _edge_leaky(a_src, a_dst):
    e, d = a_src.shape
    blk = 8000

    def kern(s_ref, t_ref, o_ref):
        x = s_ref[...] + t_ref[...]
        o_ref[...] = jnp.where(x > 0, x, 0.2 * x)

    return pl.pallas_call(
        kern,
        grid=(e // blk,),
        in_specs=[
            pl.BlockSpec((blk, d), lambda i: (i, 0)),
            pl.BlockSpec((blk, d), lambda i: (i, 0)),
        ],
        out_specs=pl.BlockSpec((blk, d), lambda i: (i, 0)),
        out_shape=jax.ShapeDtypeStruct((e, d), jnp.float32),
    )(a_src, a_dst)


def _edge_ex(el, emax_dst):
    e, d = el.shape
    blk = 8000

    def kern(e_ref, m_ref, o_ref):
        o_ref[...] = jnp.exp(e_ref[...] - m_ref[...])

    return pl.pallas_call(
        kern,
        grid=(e // blk,),
        in_specs=[
            pl.BlockSpec((blk, d), lambda i: (i, 0)),
            pl.BlockSpec((blk, d), lambda i: (i, 0)),
        ],
        out_specs=pl.BlockSpec((blk, d), lambda i: (i, 0)),
        out_shape=jax.ShapeDtypeStruct((e, d), jnp.float32),
    )(el, emax_dst)


def _gat_msg(ex, den_dst, xh_src):
    e = ex.shape[0]
    blk = 4000

    def kern(ex_ref, dn_ref, x_ref, o_ref):
        alpha = ex_ref[...] / jnp.maximum(dn_ref[...], 1e-16)
        acc = jnp.zeros((blk, _H), jnp.float32)
        for h in range(_HEADS):
            acc += alpha[:, h : h + 1] * x_ref[:, h * _H : (h + 1) * _H]
        o_ref[...] = acc * (1.0 / _HEADS)

    return pl.pallas_call(
        kern,
        grid=(e // blk,),
        in_specs=[
            pl.BlockSpec((blk, _HEADS), lambda i: (i, 0)),
            pl.BlockSpec((blk, _HEADS), lambda i: (i, 0)),
            pl.BlockSpec((blk, _HEADS * _H), lambda i: (i, 0)),
        ],
        out_specs=pl.BlockSpec((blk, _H), lambda i: (i, 0)),
        out_shape=jax.ShapeDtypeStruct((e, _H), jnp.float32),
    )(ex, den_dst, xh_src)


def _classifier(h, w1, b1, w2, b2):
    n, k = h.shape
    d1 = w1.shape[1]
    d2 = w2.shape[1]
    blk = 2000

    def kern(h_ref, w1_ref, b1_ref, w2_ref, b2_ref, o_ref):
        t = jnp.dot(h_ref[...], w1_ref[...], preferred_element_type=jnp.float32)
        t = jnp.maximum(t + b1_ref[...], 0.0)
        l = jnp.dot(t, w2_ref[...], preferred_element_type=jnp.float32) + b2_ref[...]
        m = jnp.max(l, axis=1, keepdims=True)
        lse = m + jnp.log(jnp.sum(jnp.exp(l - m), axis=1, keepdims=True))
        o_ref[...] = l - lse

    return pl.pallas_call(
        kern,
        grid=(n // blk,),
        in_specs=[
            pl.BlockSpec((blk, k), lambda i: (i, 0)),
            pl.BlockSpec((k, d1), lambda i: (0, 0)),
            pl.BlockSpec((1, d1), lambda i: (0, 0)),
            pl.BlockSpec((d1, d2), lambda i: (0, 0)),
            pl.BlockSpec((1, d2), lambda i: (0, 0)),
        ],
        out_specs=pl.BlockSpec((blk, d2), lambda i: (i, 0)),
        out_shape=jax.ShapeDtypeStruct((n, d2), jnp.float32),
    )(h, w1, b1.reshape(1, -1), w2, b2.reshape(1, -1))


def kernel(x, edge_index, W_l1, W_r1, b1, g1, be1, Wg, att_s, att_d, bg, g2, be2,
           W_l3, W_r3, b3, g3, be3, Ws, bs, Wc1, bc1, Wc2, bc2):
    src = edge_index[0]
    dst = edge_index[1]

    identity = _dense(x, Ws, bs)

    x_src = jnp.take(x, src, axis=0)
    m1 = jnp.concatenate([x_src, jnp.ones((_E, 1), jnp.float32)], axis=1)
    s1 = _scatter(dst, m1, _N, "add")
    agg1 = s1[:, :_IN] / jnp.maximum(s1[:, _IN:_IN + 1], 1.0)
    h = _dense2(agg1, W_l1, x, W_r1, b1)
    h = _bn_relu(h, g1, be1)

    xh = _dense(h, Wg, jnp.zeros((_HEADS * _H,), jnp.float32))
    eye = jnp.eye(_HEADS, dtype=jnp.float32)
    a_s_w = (att_s[:, :, None] * eye[:, None, :]).reshape(_HEADS * _H, _HEADS)
    a_d_w = (att_d[:, :, None] * eye[:, None, :]).reshape(_HEADS * _H, _HEADS)
    z4 = jnp.zeros((_HEADS,), jnp.float32)
    a_s = _dense(xh, a_s_w, z4)
    a_d = _dense(xh, a_d_w, z4)

    el = _edge_leaky(jnp.take(a_s, src, axis=0), jnp.take(a_d, dst, axis=0))
    emax = _scatter(dst, el, _N, "max")
    emax = jnp.where(jnp.isfinite(emax), emax, 0.0)
    ex = _edge_ex(el, jnp.take(emax, dst, axis=0))
    den = _scatter(dst, ex, _N, "add")
    msg = _gat_msg(ex, jnp.take(den, dst, axis=0), jnp.take(xh, src, axis=0))
    gout = _scatter(dst, msg, _N, "add") + bg.reshape(1, -1)
    h = _bn_relu(gout, g2, be2)

    h_src = jnp.take(h, src, axis=0)
    m3 = jnp.concatenate([h_src, jnp.ones((_E, 1), jnp.float32)], axis=1)
    s3 = _scatter(dst, m3, _N, "add")
    agg3 = s3[:, :_H] / jnp.maximum(s3[:, _H:_H + 1], 1.0)
    h3 = _dense2(agg3, W_l3, h, W_r3, b3)
    h3 = _bn_relu(h3, g3, be3)
    h3 = h3 + identity

    return _classifier(h3, Wc1, bc1, Wc2, bc2)


# speedup vs baseline: 3.6520x; 1.4407x over previous
import jax
import jax.numpy as jnp
from jax.experimental import pallas as pl
from jax.experimental.pallas import tpu as pltpu

_N = 50000
_E = 800000
_IN = 20
_H = 64
_HEADS = 4

_ARB = pltpu.CompilerParams(dimension_semantics=("arbitrary",))


def _scatter(dst, m, n_rows, op):
    e, d = m.shape
    chunk = 256
    init = 0.0 if op == "add" else -jnp.inf

    def kern(dst_ref, m_ref, out_ref):
        @pl.when(pl.program_id(0) == 0)
        def _():
            out_ref[...] = jnp.full_like(out_ref, init)

        def body(j, carry):
            r = dst_ref[j]
            row = m_ref[pl.ds(j, 1), :]
            cur = out_ref[pl.ds(r, 1), :]
            new = cur + row if op == "add" else jnp.maximum(cur, row)
            out_ref[pl.ds(r, 1), :] = new
            return carry

        jax.lax.fori_loop(0, chunk, body, 0, unroll=8)

    return pl.pallas_call(
        kern,
        grid=(e // chunk,),
        in_specs=[
            pl.BlockSpec((chunk,), lambda i: (i,), memory_space=pltpu.SMEM),
            pl.BlockSpec((chunk, d), lambda i: (i, 0)),
        ],
        out_specs=pl.BlockSpec((n_rows, d), lambda i: (0, 0)),
        out_shape=jax.ShapeDtypeStruct((n_rows, d), jnp.float32),
        compiler_params=_ARB,
    )(dst, m)


def _dense(a, w, b, act=None):
    n, k = a.shape
    h = w.shape[1]
    blk = 2000

    def kern(a_ref, w_ref, b_ref, o_ref):
        o = jnp.dot(a_ref[...], w_ref[...], preferred_element_type=jnp.float32)
        o = o + b_ref[...]
        if act == "relu":
            o = jnp.maximum(o, 0.0)
        o_ref[...] = o

    return pl.pallas_call(
        kern,
        grid=(n // blk,),
        in_specs=[
            pl.BlockSpec((blk, k), lambda i: (i, 0)),
            pl.BlockSpec((k, h), lambda i: (0, 0)),
            pl.BlockSpec((1, h), lambda i: (0, 0)),
        ],
        out_specs=pl.BlockSpec((blk, h), lambda i: (i, 0)),
        out_shape=jax.ShapeDtypeStruct((n, h), jnp.float32),
    )(a, w, b.reshape(1, -1))


def _dense2(a, wa, c, wc, b):
    n, ka = a.shape
    kc = c.shape[1]
    h = wa.shape[1]
    blk = 2000

    def kern(a_ref, wa_ref, c_ref, wc_ref, b_ref, o_ref):
        o = jnp.dot(a_ref[...], wa_ref[...], preferred_element_type=jnp.float32)
        o = o + jnp.dot(c_ref[...], wc_ref[...], preferred_element_type=jnp.float32)
        o_ref[...] = o + b_ref[...]

    return pl.pallas_call(
        kern,
        grid=(n // blk,),
        in_specs=[
            pl.BlockSpec((blk, ka), lambda i: (i, 0)),
            pl.BlockSpec((ka, h), lambda i: (0, 0)),
            pl.BlockSpec((blk, kc), lambda i: (i, 0)),
            pl.BlockSpec((kc, h), lambda i: (0, 0)),
            pl.BlockSpec((1, h), lambda i: (0, 0)),
        ],
        out_specs=pl.BlockSpec((blk, h), lambda i: (i, 0)),
        out_shape=jax.ShapeDtypeStruct((n, h), jnp.float32),
    )(a, wa, c, wc, b.reshape(1, -1))


def _bn_stats(h):
    n, d = h.shape
    blk = 2000

    def kern(h_ref, s_ref, q_ref):
        @pl.when(pl.program_id(0) == 0)
        def _():
            s_ref[...] = jnp.zeros_like(s_ref)
            q_ref[...] = jnp.zeros_like(q_ref)

        x = h_ref[...]
        s_ref[...] += jnp.sum(x, axis=0, keepdims=True)
        q_ref[...] += jnp.sum(x * x, axis=0, keepdims=True)

    s, q = pl.pallas_call(
        kern,
        grid=(n // blk,),
        in_specs=[pl.BlockSpec((blk, d), lambda i: (i, 0))],
        out_specs=[pl.BlockSpec((1, d), lambda i: (0, 0))] * 2,
        out_shape=[jax.ShapeDtypeStruct((1, d), jnp.float32)] * 2,
        compiler_params=_ARB,
    )(h)
    m = s / n
    v = q / n - m * m
    return m, v


def _affine_relu(h, scale, shift):
    n, d = h.shape
    blk = 2000

    def kern(h_ref, sc_ref, sh_ref, o_ref):
        o_ref[...] = jnp.maximum(h_ref[...] * sc_ref[...] + sh_ref[...], 0.0)

    return pl.pallas_call(
        kern,
        grid=(n // blk,),
        in_specs=[
            pl.BlockSpec((blk, d), lambda i: (i, 0)),
            pl.BlockSpec((1, d), lambda i: (0, 0)),
            pl.BlockSpec((1, d), lambda i: (0, 0)),
        ],
        out_specs=pl.BlockSpec((blk, d), lambda i: (i, 0)),
        out_shape=jax.ShapeDtypeStruct((n, d), jnp.float32),
    )(h, scale, shift)


def _bn_relu(h, g, be):
    m, v = _bn_stats(h)
    scale = g.reshape(1, -1) / jnp.sqrt(v + 1e-5)
    shift = be.reshape(1, -1) - m * scale
    return _affine_relu(h, scale, shift)


def _edge_leaky(a_src, a_dst):
    e, d = a_src.shape
    blk = 8000

    def kern(s_ref, t_ref, o_ref):
        x = s_ref[...] + t_ref[...]
        o_ref[...] = jnp.where(x > 0, x, 0.2 * x)

    return pl.pallas_call(
        kern,
        grid=(e // blk,),
        in_specs=[
            pl.BlockSpec((blk, d), lambda i: (i, 0)),
            pl.BlockSpec((blk, d), lambda i: (i, 0)),
        ],
        out_specs=pl.BlockSpec((blk, d), lambda i: (i, 0)),
        out_shape=jax.ShapeDtypeStruct((e, d), jnp.float32),
    )(a_src, a_dst)


def _edge_ex(el, emax_dst):
    e, d = el.shape
    blk = 8000

    def kern(e_ref, m_ref, o_ref):
        o_ref[...] = jnp.exp(e_ref[...] - m_ref[...])

    return pl.pallas_call(
        kern,
        grid=(e // blk,),
        in_specs=[
            pl.BlockSpec((blk, d), lambda i: (i, 0)),
            pl.BlockSpec((blk, d), lambda i: (i, 0)),
        ],
        out_specs=pl.BlockSpec((blk, d), lambda i: (i, 0)),
        out_shape=jax.ShapeDtypeStruct((e, d), jnp.float32),
    )(el, emax_dst)


def _gat_msg(ex, den_dst, xh_src):
    e = ex.shape[0]
    blk = 4000

    def kern(ex_ref, dn_ref, x_ref, o_ref):
        alpha = ex_ref[...] / jnp.maximum(dn_ref[...], 1e-16)
        acc = jnp.zeros((blk, _H), jnp.float32)
        for h in range(_HEADS):
            acc += alpha[:, h : h + 1] * x_ref[:, h * _H : (h + 1) * _H]
        o_ref[...] = acc * (1.0 / _HEADS)

    return pl.pallas_call(
        kern,
        grid=(e // blk,),
        in_specs=[
            pl.BlockSpec((blk, _HEADS), lambda i: (i, 0)),
            pl.BlockSpec((blk, _HEADS), lambda i: (i, 0)),
            pl.BlockSpec((blk, _HEADS * _H), lambda i: (i, 0)),
        ],
        out_specs=pl.BlockSpec((blk, _H), lambda i: (i, 0)),
        out_shape=jax.ShapeDtypeStruct((e, _H), jnp.float32),
    )(ex, den_dst, xh_src)


def _classifier(h, w1, b1, w2, b2):
    n, k = h.shape
    d1 = w1.shape[1]
    d2 = w2.shape[1]
    blk = 2000

    def kern(h_ref, w1_ref, b1_ref, w2_ref, b2_ref, o_ref):
        t = jnp.dot(h_ref[...], w1_ref[...], preferred_element_type=jnp.float32)
        t = jnp.maximum(t + b1_ref[...], 0.0)
        l = jnp.dot(t, w2_ref[...], preferred_element_type=jnp.float32) + b2_ref[...]
        m = jnp.max(l, axis=1, keepdims=True)
        lse = m + jnp.log(jnp.sum(jnp.exp(l - m), axis=1, keepdims=True))
        o_ref[...] = l - lse

    return pl.pallas_call(
        kern,
        grid=(n // blk,),
        in_specs=[
            pl.BlockSpec((blk, k), lambda i: (i, 0)),
            pl.BlockSpec((k, d1), lambda i: (0, 0)),
            pl.BlockSpec((1, d1), lambda i: (0, 0)),
            pl.BlockSpec((d1, d2), lambda i: (0, 0)),
            pl.BlockSpec((1, d2), lambda i: (0, 0)),
        ],
        out_specs=pl.BlockSpec((blk, d2), lambda i: (i, 0)),
        out_shape=jax.ShapeDtypeStruct((n, d2), jnp.float32),
    )(h, w1, b1.reshape(1, -1), w2, b2.reshape(1, -1))


def kernel(x, edge_index, W_l1, W_r1, b1, g1, be1, Wg, att_s, att_d, bg, g2, be2,
           W_l3, W_r3, b3, g3, be3, Ws, bs, Wc1, bc1, Wc2, bc2):
    src = edge_index[0]
    dst = edge_index[1]

    identity = _dense(x, Ws, bs)

    x_src = jnp.take(x, src, axis=0)
    m1 = jnp.concatenate([x_src, jnp.ones((_E, 1), jnp.float32)], axis=1)
    s1 = _scatter(dst, m1, _N, "add")
    agg1 = s1[:, :_IN] / jnp.maximum(s1[:, _IN:_IN + 1], 1.0)
    h = _dense2(agg1, W_l1, x, W_r1, b1)
    h = _bn_relu(h, g1, be1)

    xh = _dense(h, Wg, jnp.zeros((_HEADS * _H,), jnp.float32))
    eye = jnp.eye(_HEADS, dtype=jnp.float32)
    a_s_w = (att_s[:, :, None] * eye[:, None, :]).reshape(_HEADS * _H, _HEADS)
    a_d_w = (att_d[:, :, None] * eye[:, None, :]).reshape(_HEADS * _H, _HEADS)
    z4 = jnp.zeros((_HEADS,), jnp.float32)
    a_s = _dense(xh, a_s_w, z4)
    a_d = _dense(xh, a_d_w, z4)

    el = _edge_leaky(jnp.take(a_s, src, axis=0), jnp.take(a_d, dst, axis=0))
    emax = _scatter(dst, el, _N, "max")
    emax = jnp.where(jnp.isfinite(emax), emax, 0.0)
    ex = _edge_ex(el, jnp.take(emax, dst, axis=0))
    den = _scatter(dst, ex, _N, "add")
    msg = _gat_msg(ex, jnp.take(den, dst, axis=0), jnp.take(xh, src, axis=0))
    gout = _scatter(dst, msg, _N, "add") + bg.reshape(1, -1)
    h = _bn_relu(gout, g2, be2)

    h_src = jnp.take(h, src, axis=0)
    m3 = jnp.concatenate([h_src, jnp.ones((_E, 1), jnp.float32)], axis=1)
    s3 = _scatter(dst, m3, _N, "add")
    agg3 = s3[:, :_H] / jnp.maximum(s3[:, _H:_H + 1], 1.0)
    h3 = _dense2(agg3, W_l3, h, W_r3, b3)
    h3 = _bn_relu(h3, g3, be3)
    h3 = h3 + identity

    return _classifier(h3, Wc1, bc1, Wc2, bc2)
